# baseline (speedup 1.0000x reference)
import jax
import jax.numpy as jnp
from jax import lax
from jax.experimental import pallas as pl
from jax.experimental.pallas import tpu as pltpu
from jax.experimental.pallas import tpu_sc as plsc

VOCAB = 1000000
EMBED_DIM = 64
BATCH = 16384
CTX = 20

NC = 2
NS = 16
LANES = 16
NW = NC * NS
BPW = BATCH // NW
CB = 32
NCH = BPW // CB
IPC = CB * CTX
IDX_W = 128
NSTR = IPC // IDX_W
VPD = EMBED_DIM // LANES
IROWS = BPW * CTX // IDX_W


def _cbow_body(idx_hbm, table_hbm, out_hbm, idx_v, loff_v, rows_v, out_v, sem):
    wid = lax.axis_index("s") * NC + lax.axis_index("c")
    inv_ctx = jnp.float32(1.0 / CTX)

    pltpu.sync_copy(idx_hbm.at[pl.ds(wid * IROWS, IROWS)], idx_v)

    def prep_body(r, _):
        for g in range(IDX_W // LANES):
            sl = pl.ds(g * LANES, LANES)
            v = idx_v[r, sl]
            loff_v[pl.ds(r * IDX_W + g * LANES, LANES)] = lax.shift_left(
                lax.bitwise_and(v, jnp.int32(1)), jnp.int32(6)
            )
            idx_v[r, sl] = lax.shift_right_logical(v, jnp.int32(1))
        return 0

    lax.fori_loop(0, IROWS, prep_body, 0)

    def chunk_body(c, _):
        copies = [
            pltpu.async_copy(
                table_hbm.at[idx_v.at[c * NSTR + k]],
                rows_v.at[pl.ds(k * IDX_W, IDX_W)],
                sem,
            )
            for k in range(NSTR)
        ]
        for cp in copies:
            cp.wait()

        def item_body(i, _):
            base = i * CTX
            gbase = c * NSTR * IDX_W + base
            acc = [None] * VPD
            for j in range(CTX):
                lo = loff_v[pl.ds(gbase + j, LANES)][0]
                for q in range(VPD):
                    qoff = q * LANES
                    x = rows_v[base + j, pl.ds(lo + qoff, LANES)]
                    acc[q] = x if acc[q] is None else acc[q] + x
            for q in range(VPD):
                out_v[i, pl.ds(q * LANES, LANES)] = acc[q] * inv_ctx
            return 0

        lax.fori_loop(0, CB, item_body, 0)

        out_row0 = wid * BPW + c * CB
        pltpu.sync_copy(out_v, out_hbm.at[pl.ds(out_row0, CB)])
        return 0

    lax.fori_loop(0, NCH, chunk_body, 0)


@jax.jit
def _cbow(idx2d, table2):
    mesh = plsc.VectorSubcoreMesh(
        core_axis_name="c", subcore_axis_name="s", num_cores=NC, num_subcores=NS
    )
    return pl.kernel(
        _cbow_body,
        out_type=jax.ShapeDtypeStruct((BATCH, EMBED_DIM), jnp.float32),
        mesh=mesh,
        scratch_types=[
            pltpu.VMEM((IROWS, IDX_W), jnp.int32),
            pltpu.VMEM((IROWS * IDX_W + LANES,), jnp.int32),
            pltpu.VMEM((IPC, 2 * EMBED_DIM), jnp.float32),
            pltpu.VMEM((CB, EMBED_DIM), jnp.float32),
            pltpu.SemaphoreType.DMA,
        ],
        compiler_params=pltpu.CompilerParams(use_tc_tiling_on_sc=False),
        name="cbow_sc",
    )(idx2d, table2)


def kernel(context_idxs, embeddings):
    idx2d = context_idxs.astype(jnp.int32).reshape(BATCH * CTX // IDX_W, IDX_W)
    table2 = embeddings.reshape(VOCAB // 2, 2 * EMBED_DIM)
    return _cbow(idx2d, table2)


# speedup vs baseline: 1.2302x; 1.2302x over previous
import jax
import jax.numpy as jnp
from jax import lax
from jax.experimental import pallas as pl
from jax.experimental.pallas import tpu as pltpu
from jax.experimental.pallas import tpu_sc as plsc

VOCAB = 1000000
EMBED_DIM = 64
BATCH = 16384
CTX = 20

NC = 2
NS = 16
LANES = 16
NW = NC * NS
BPW = BATCH // NW
CB = 32
NCH = BPW // CB
IPC = CB * CTX
IDX_W = 128
NSTR = IPC // IDX_W
VPD = EMBED_DIM // LANES
IROWS = BPW * CTX // IDX_W


def _cbow_body(idx_hbm, table_hbm, out_hbm, idx_v, loff_v, rows_v, out_v, sem):
    wid = lax.axis_index("s") * NC + lax.axis_index("c")
    inv_ctx = jnp.float32(1.0 / CTX)

    pltpu.sync_copy(idx_hbm.at[pl.ds(wid * IROWS, IROWS)], idx_v)

    def prep_body(r, _):
        for g in range(IDX_W // LANES):
            sl = pl.ds(g * LANES, LANES)
            v = idx_v[r, sl]
            loff_v[pl.ds(r * IDX_W + g * LANES, LANES)] = lax.shift_left(
                lax.bitwise_and(lax.shift_right_logical(v, jnp.int32(10)), jnp.int32(1)),
                jnp.int32(6),
            )
            idx_v[r, sl] = lax.bitwise_or(
                lax.shift_left(lax.shift_right_logical(v, jnp.int32(11)), jnp.int32(10)),
                lax.bitwise_and(v, jnp.int32(1023)),
            )
        return 0

    lax.fori_loop(0, IROWS, prep_body, 0)

    def chunk_body(c, _):
        copies = [
            pltpu.async_copy(
                table_hbm.at[idx_v.at[c * NSTR + k]],
                rows_v.at[pl.ds(k * IDX_W, IDX_W)],
                sem,
            )
            for k in range(NSTR)
        ]
        for cp in copies:
            cp.wait()

        def item_body(i, _):
            base = i * CTX
            gbase = c * NSTR * IDX_W + base
            acc = [None] * VPD
            for j in range(CTX):
                lo = loff_v[pl.ds(gbase + j, LANES)][0]
                for q in range(VPD):
                    qoff = q * LANES
                    x = rows_v[base + j, pl.ds(lo + qoff, LANES)]
                    acc[q] = x if acc[q] is None else acc[q] + x
            for q in range(VPD):
                out_v[i, pl.ds(q * LANES, LANES)] = acc[q] * inv_ctx
            return 0

        lax.fori_loop(0, CB, item_body, 0)

        out_row0 = wid * BPW + c * CB
        pltpu.sync_copy(out_v, out_hbm.at[pl.ds(out_row0, CB)])
        return 0

    lax.fori_loop(0, NCH, chunk_body, 0)


TNV = 2048
TGRID = (VOCAB + TNV - 1) // TNV
PACKV = TGRID * TNV // 2


def _transpose_block(xT_ref, out_ref):
    xT = xT_ref[...]
    rows = xT.T
    out_ref[...] = jnp.concatenate([rows[: TNV // 2], rows[TNV // 2:]], axis=1)


@jax.jit
def _transpose(tableT):
    return pl.pallas_call(
        _transpose_block,
        grid=(TGRID,),
        in_specs=[pl.BlockSpec((EMBED_DIM, TNV), lambda i: (0, i))],
        out_specs=pl.BlockSpec((TNV // 2, 2 * EMBED_DIM), lambda i: (i, 0)),
        out_shape=jax.ShapeDtypeStruct((PACKV, 2 * EMBED_DIM), jnp.float32),
        name="cbow_transpose",
    )(tableT)


@jax.jit
def _cbow(idx2d, table2):
    mesh = plsc.VectorSubcoreMesh(
        core_axis_name="c", subcore_axis_name="s", num_cores=NC, num_subcores=NS
    )
    return pl.kernel(
        _cbow_body,
        out_type=jax.ShapeDtypeStruct((BATCH, EMBED_DIM), jnp.float32),
        mesh=mesh,
        scratch_types=[
            pltpu.VMEM((IROWS, IDX_W), jnp.int32),
            pltpu.VMEM((IROWS * IDX_W + LANES,), jnp.int32),
            pltpu.VMEM((IPC, 2 * EMBED_DIM), jnp.float32),
            pltpu.VMEM((CB, EMBED_DIM), jnp.float32),
            pltpu.SemaphoreType.DMA,
        ],
        compiler_params=pltpu.CompilerParams(use_tc_tiling_on_sc=False),
        name="cbow_sc",
    )(idx2d, table2)


def kernel(context_idxs, embeddings):
    idx2d = context_idxs.astype(jnp.int32).reshape(BATCH * CTX // IDX_W, IDX_W)
    table2 = _transpose(embeddings.T)
    return _cbow(idx2d, table2)


# speedup vs baseline: 1.2321x; 1.0015x over previous
import jax
import jax.numpy as jnp
from jax import lax
from jax.experimental import pallas as pl
from jax.experimental.pallas import tpu as pltpu
from jax.experimental.pallas import tpu_sc as plsc

VOCAB = 1000000
EMBED_DIM = 64
BATCH = 16384
CTX = 20

NC = 2
NS = 16
LANES = 16
NW = NC * NS
BPW = BATCH // NW
CB = 32
NCH = BPW // CB
IPC = CB * CTX
IDX_W = 128
NSTR = IPC // IDX_W
VPD = EMBED_DIM // LANES
IROWS = BPW * CTX // IDX_W


def _cbow_body(idx_hbm, table_hbm, out_hbm, idx_v, loff_v, rows_v, out_v, sem):
    wid = lax.axis_index("s") * NC + lax.axis_index("c")
    inv_ctx = jnp.float32(1.0 / CTX)

    pltpu.sync_copy(idx_hbm.at[pl.ds(wid * IROWS, IROWS)], idx_v)

    def prep_body(r, _):
        for g in range(IDX_W // LANES):
            sl = pl.ds(g * LANES, LANES)
            v = idx_v[r, sl]
            loff_v[pl.ds(r * IDX_W + g * LANES, LANES)] = lax.shift_left(
                lax.bitwise_and(lax.shift_right_logical(v, jnp.int32(10)), jnp.int32(1)),
                jnp.int32(6),
            )
            idx_v[r, sl] = lax.bitwise_or(
                lax.shift_left(lax.shift_right_logical(v, jnp.int32(11)), jnp.int32(10)),
                lax.bitwise_and(v, jnp.int32(1023)),
            )
        return 0

    lax.fori_loop(0, IROWS, prep_body, 0)

    def chunk_body(c, _):
        copies = [
            pltpu.async_copy(
                table_hbm.at[idx_v.at[c * NSTR + k]],
                rows_v.at[pl.ds(k * IDX_W, IDX_W)],
                sem,
            )
            for k in range(NSTR)
        ]
        for cp in copies:
            cp.wait()

        def item_body(i, _):
            base = i * CTX
            gbase = c * NSTR * IDX_W + base
            acc = [None] * VPD
            for j in range(CTX):
                lo = loff_v[pl.ds(gbase + j, LANES)][0]
                for q in range(VPD):
                    qoff = q * LANES
                    x = rows_v[base + j, pl.ds(lo + qoff, LANES)]
                    acc[q] = x if acc[q] is None else acc[q] + x
            for q in range(VPD):
                out_v[i, pl.ds(q * LANES, LANES)] = acc[q] * inv_ctx
            return 0

        lax.fori_loop(0, CB, item_body, 0)

        out_row0 = wid * BPW + c * CB
        pltpu.sync_copy(out_v, out_hbm.at[pl.ds(out_row0, CB)])
        return 0

    lax.fori_loop(0, NCH, chunk_body, 0)


TNV = 2048
TGRID = (VOCAB + TNV - 1) // TNV
PACKV = TGRID * TNV // 2


def _transpose_block(xT_ref, out_ref):
    xT = xT_ref[...]
    rows = xT.T
    out_ref[...] = jnp.concatenate([rows[: TNV // 2], rows[TNV // 2:]], axis=1)


@jax.jit
def _transpose(tableT):
    return pl.pallas_call(
        _transpose_block,
        grid=(TGRID,),
        in_specs=[pl.BlockSpec((EMBED_DIM, TNV), lambda i: (0, i))],
        out_specs=pl.BlockSpec((TNV // 2, 2 * EMBED_DIM), lambda i: (i, 0)),
        out_shape=jax.ShapeDtypeStruct((PACKV, 2 * EMBED_DIM), jnp.float32),
        compiler_params=pltpu.CompilerParams(
            dimension_semantics=("parallel",)
        ),
        name="cbow_transpose",
    )(tableT)


@jax.jit
def _cbow(idx2d, table2):
    mesh = plsc.VectorSubcoreMesh(
        core_axis_name="c", subcore_axis_name="s", num_cores=NC, num_subcores=NS
    )
    return pl.kernel(
        _cbow_body,
        out_type=jax.ShapeDtypeStruct((BATCH, EMBED_DIM), jnp.float32),
        mesh=mesh,
        scratch_types=[
            pltpu.VMEM((IROWS, IDX_W), jnp.int32),
            pltpu.VMEM((IROWS * IDX_W + LANES,), jnp.int32),
            pltpu.VMEM((IPC, 2 * EMBED_DIM), jnp.float32),
            pltpu.VMEM((CB, EMBED_DIM), jnp.float32),
            pltpu.SemaphoreType.DMA,
        ],
        compiler_params=pltpu.CompilerParams(use_tc_tiling_on_sc=False),
        name="cbow_sc",
    )(idx2d, table2)


def kernel(context_idxs, embeddings):
    idx2d = context_idxs.astype(jnp.int32).reshape(BATCH * CTX // IDX_W, IDX_W)
    table2 = _transpose(embeddings.T)
    return _cbow(idx2d, table2)


# speedup vs baseline: 1.7996x; 1.4606x over previous
import jax
import jax.numpy as jnp
from jax import lax
from jax.experimental import pallas as pl
from jax.experimental.pallas import tpu as pltpu
from jax.experimental.pallas import tpu_sc as plsc

VOCAB = 1000000
EMBED_DIM = 64
BATCH = 16384
CTX = 20

NC = 2
NS = 16
LANES = 16
NW = NC * NS
BPW = BATCH // NW
CB = 32
NCH = BPW // CB
IPC = CB * CTX
IDX_W = 128
NSTR = IPC // IDX_W
VPD = EMBED_DIM // LANES
IROWS = BPW * CTX // IDX_W

TNV = 8192
HB = 12
TGRID = (VOCAB + TNV - 1) // TNV
PACKV = TGRID * TNV // 2


def _cbow_body(idx_hbm, table_hbm, out_hbm, idx_v, loff_v, rows_v, out_v, sem):
    wid = lax.axis_index("s") * NC + lax.axis_index("c")
    inv_ctx = jnp.float32(1.0 / CTX)

    pltpu.sync_copy(idx_hbm.at[pl.ds(wid * IROWS, IROWS)], idx_v)

    def prep_body(r, _):
        for g in range(IDX_W // LANES):
            sl = pl.ds(g * LANES, LANES)
            v = idx_v[r, sl]
            loff_v[pl.ds(r * IDX_W + g * LANES, LANES)] = lax.shift_left(
                lax.bitwise_and(lax.shift_right_logical(v, jnp.int32(HB)), jnp.int32(1)),
                jnp.int32(6),
            )
            idx_v[r, sl] = lax.bitwise_or(
                lax.shift_left(lax.shift_right_logical(v, jnp.int32(HB + 1)), jnp.int32(HB)),
                lax.bitwise_and(v, jnp.int32(TNV // 2 - 1)),
            )
        return 0

    lax.fori_loop(0, IROWS, prep_body, 0)

    def chunk_body(c, _):
        copies = [
            pltpu.async_copy(
                table_hbm.at[idx_v.at[c * NSTR + k]],
                rows_v.at[pl.ds(k * IDX_W, IDX_W)],
                sem,
            )
            for k in range(NSTR)
        ]
        for cp in copies:
            cp.wait()

        def item_body(i, _):
            base = i * CTX
            gbase = c * NSTR * IDX_W + base
            acc = [None] * VPD
            for j in range(CTX):
                lo = loff_v[pl.ds(gbase + j, LANES)][0]
                for q in range(VPD):
                    qoff = q * LANES
                    x = rows_v[base + j, pl.ds(lo + qoff, LANES)]
                    acc[q] = x if acc[q] is None else acc[q] + x
            for q in range(VPD):
                out_v[i, pl.ds(q * LANES, LANES)] = acc[q] * inv_ctx
            return 0

        lax.fori_loop(0, CB, item_body, 0)

        out_row0 = wid * BPW + c * CB
        pltpu.sync_copy(out_v, out_hbm.at[pl.ds(out_row0, CB)])
        return 0

    lax.fori_loop(0, NCH, chunk_body, 0)


def _transpose_block(xT_ref, out_ref):
    xT = xT_ref[...]
    rows = xT.T
    out_ref[...] = jnp.concatenate([rows[: TNV // 2], rows[TNV // 2:]], axis=1)


@jax.jit
def _transpose(tableT):
    return pl.pallas_call(
        _transpose_block,
        grid=(TGRID,),
        in_specs=[pl.BlockSpec((EMBED_DIM, TNV), lambda i: (0, i))],
        out_specs=pl.BlockSpec((TNV // 2, 2 * EMBED_DIM), lambda i: (i, 0)),
        out_shape=jax.ShapeDtypeStruct((PACKV, 2 * EMBED_DIM), jnp.float32),
        compiler_params=pltpu.CompilerParams(
            dimension_semantics=("parallel",)
        ),
        name="cbow_transpose",
    )(tableT)


@jax.jit
def _cbow(idx2d, table2):
    mesh = plsc.VectorSubcoreMesh(
        core_axis_name="c", subcore_axis_name="s", num_cores=NC, num_subcores=NS
    )
    return pl.kernel(
        _cbow_body,
        out_type=jax.ShapeDtypeStruct((BATCH, EMBED_DIM), jnp.float32),
        mesh=mesh,
        scratch_types=[
            pltpu.VMEM((IROWS, IDX_W), jnp.int32),
            pltpu.VMEM((IROWS * IDX_W + LANES,), jnp.int32),
            pltpu.VMEM((IPC, 2 * EMBED_DIM), jnp.float32),
            pltpu.VMEM((CB, EMBED_DIM), jnp.float32),
            pltpu.SemaphoreType.DMA,
        ],
        compiler_params=pltpu.CompilerParams(use_tc_tiling_on_sc=False),
        name="cbow_sc",
    )(idx2d, table2)


def kernel(context_idxs, embeddings):
    idx2d = context_idxs.astype(jnp.int32).reshape(BATCH * CTX // IDX_W, IDX_W)
    table2 = _transpose(embeddings.T)
    return _cbow(idx2d, table2)


# speedup vs baseline: 1.9589x; 1.0885x over previous
import jax
import jax.numpy as jnp
from jax import lax
from jax.experimental import pallas as pl
from jax.experimental.pallas import tpu as pltpu
from jax.experimental.pallas import tpu_sc as plsc

VOCAB = 1000000
EMBED_DIM = 64
BATCH = 16384
CTX = 20

NC = 2
NS = 16
LANES = 16
NW = NC * NS
BPW = BATCH // NW
CB = 32
NCH = BPW // CB
IPC = CB * CTX
IDX_W = 128
NSTR = IPC // IDX_W
VPD = EMBED_DIM // LANES
IROWS = BPW * CTX // IDX_W

TNV = 16384
HB = 13
TGRID = (VOCAB + TNV - 1) // TNV
PACKV = TGRID * TNV // 2


def _cbow_body(idx_hbm, table_hbm, out_hbm, idx_v, loff_v, rows_v, out_v, sem):
    wid = lax.axis_index("s") * NC + lax.axis_index("c")
    inv_ctx = jnp.float32(1.0 / CTX)

    pltpu.sync_copy(idx_hbm.at[pl.ds(wid * IROWS, IROWS)], idx_v)

    def prep_body(r, _):
        for g in range(IDX_W // LANES):
            sl = pl.ds(g * LANES, LANES)
            v = idx_v[r, sl]
            loff_v[pl.ds(r * IDX_W + g * LANES, LANES)] = lax.shift_left(
                lax.bitwise_and(lax.shift_right_logical(v, jnp.int32(HB)), jnp.int32(1)),
                jnp.int32(6),
            )
            idx_v[r, sl] = lax.bitwise_or(
                lax.shift_left(lax.shift_right_logical(v, jnp.int32(HB + 1)), jnp.int32(HB)),
                lax.bitwise_and(v, jnp.int32(TNV // 2 - 1)),
            )
        return 0

    lax.fori_loop(0, IROWS, prep_body, 0)

    def chunk_body(c, _):
        copies = [
            pltpu.async_copy(
                table_hbm.at[idx_v.at[c * NSTR + k]],
                rows_v.at[pl.ds(k * IDX_W, IDX_W)],
                sem,
            )
            for k in range(NSTR)
        ]
        for cp in copies:
            cp.wait()

        def item_body(i, _):
            base = i * CTX
            gbase = c * NSTR * IDX_W + base
            acc = [None] * VPD
            for j in range(CTX):
                lo = loff_v[pl.ds(gbase + j, LANES)][0]
                for q in range(VPD):
                    qoff = q * LANES
                    x = rows_v[base + j, pl.ds(lo + qoff, LANES)]
                    acc[q] = x if acc[q] is None else acc[q] + x
            for q in range(VPD):
                out_v[i, pl.ds(q * LANES, LANES)] = acc[q] * inv_ctx
            return 0

        lax.fori_loop(0, CB, item_body, 0)

        out_row0 = wid * BPW + c * CB
        pltpu.sync_copy(out_v, out_hbm.at[pl.ds(out_row0, CB)])
        return 0

    lax.fori_loop(0, NCH, chunk_body, 0)


def _transpose_block(xT_ref, out_ref):
    xT = xT_ref[...]
    rows = xT.T
    out_ref[...] = jnp.concatenate([rows[: TNV // 2], rows[TNV // 2:]], axis=1)


@jax.jit
def _transpose(tableT):
    return pl.pallas_call(
        _transpose_block,
        grid=(TGRID,),
        in_specs=[pl.BlockSpec((EMBED_DIM, TNV), lambda i: (0, i))],
        out_specs=pl.BlockSpec((TNV // 2, 2 * EMBED_DIM), lambda i: (i, 0)),
        out_shape=jax.ShapeDtypeStruct((PACKV, 2 * EMBED_DIM), jnp.float32),
        compiler_params=pltpu.CompilerParams(
            dimension_semantics=("parallel",)
        ),
        name="cbow_transpose",
    )(tableT)


@jax.jit
def _cbow(idx2d, table2):
    mesh = plsc.VectorSubcoreMesh(
        core_axis_name="c", subcore_axis_name="s", num_cores=NC, num_subcores=NS
    )
    return pl.kernel(
        _cbow_body,
        out_type=jax.ShapeDtypeStruct((BATCH, EMBED_DIM), jnp.float32),
        mesh=mesh,
        scratch_types=[
            pltpu.VMEM((IROWS, IDX_W), jnp.int32),
            pltpu.VMEM((IROWS * IDX_W + LANES,), jnp.int32),
            pltpu.VMEM((IPC, 2 * EMBED_DIM), jnp.float32),
            pltpu.VMEM((CB, EMBED_DIM), jnp.float32),
            pltpu.SemaphoreType.DMA,
        ],
        compiler_params=pltpu.CompilerParams(use_tc_tiling_on_sc=False),
        name="cbow_sc",
    )(idx2d, table2)


def kernel(context_idxs, embeddings):
    idx2d = context_idxs.astype(jnp.int32).reshape(BATCH * CTX // IDX_W, IDX_W)
    table2 = _transpose(embeddings.T)
    return _cbow(idx2d, table2)


# speedup vs baseline: 2.0326x; 1.0376x over previous
import jax
import jax.numpy as jnp
from jax import lax
from jax.experimental import pallas as pl
from jax.experimental.pallas import tpu as pltpu
from jax.experimental.pallas import tpu_sc as plsc

VOCAB = 1000000
EMBED_DIM = 64
BATCH = 16384
CTX = 20

NC = 2
NS = 16
LANES = 16
NW = NC * NS
BPW = BATCH // NW
CB = 32
NCH = BPW // CB
IPC = CB * CTX
IDX_W = 128
NSTR = IPC // IDX_W
VPD = EMBED_DIM // LANES
IROWS = BPW * CTX // IDX_W

TNV = 32768
HB = 14
TGRID = (VOCAB + TNV - 1) // TNV
PACKV = TGRID * TNV // 2


def _cbow_body(idx_hbm, table_hbm, out_hbm, idx_v, loff_v, rows_v, out_v, sem):
    wid = lax.axis_index("s") * NC + lax.axis_index("c")
    inv_ctx = jnp.float32(1.0 / CTX)

    pltpu.sync_copy(idx_hbm.at[pl.ds(wid * IROWS, IROWS)], idx_v)

    def prep_body(r, _):
        for g in range(IDX_W // LANES):
            sl = pl.ds(g * LANES, LANES)
            v = idx_v[r, sl]
            loff_v[pl.ds(r * IDX_W + g * LANES, LANES)] = lax.shift_left(
                lax.bitwise_and(lax.shift_right_logical(v, jnp.int32(HB)), jnp.int32(1)),
                jnp.int32(6),
            )
            idx_v[r, sl] = lax.bitwise_or(
                lax.shift_left(lax.shift_right_logical(v, jnp.int32(HB + 1)), jnp.int32(HB)),
                lax.bitwise_and(v, jnp.int32(TNV // 2 - 1)),
            )
        return 0

    lax.fori_loop(0, IROWS, prep_body, 0)

    def chunk_body(c, _):
        copies = [
            pltpu.async_copy(
                table_hbm.at[idx_v.at[c * NSTR + k]],
                rows_v.at[pl.ds(k * IDX_W, IDX_W)],
                sem,
            )
            for k in range(NSTR)
        ]
        for cp in copies:
            cp.wait()

        def item_body(i, _):
            base = i * CTX
            gbase = c * NSTR * IDX_W + base
            acc = [None] * VPD
            for j in range(CTX):
                lo = loff_v[pl.ds(gbase + j, LANES)][0]
                for q in range(VPD):
                    qoff = q * LANES
                    x = rows_v[base + j, pl.ds(lo + qoff, LANES)]
                    acc[q] = x if acc[q] is None else acc[q] + x
            for q in range(VPD):
                out_v[i, pl.ds(q * LANES, LANES)] = acc[q] * inv_ctx
            return 0

        lax.fori_loop(0, CB, item_body, 0)

        out_row0 = wid * BPW + c * CB
        pltpu.sync_copy(out_v, out_hbm.at[pl.ds(out_row0, CB)])
        return 0

    lax.fori_loop(0, NCH, chunk_body, 0)


def _transpose_block(xT_ref, out_ref):
    xT = xT_ref[...]
    rows = xT.T
    out_ref[...] = jnp.concatenate([rows[: TNV // 2], rows[TNV // 2:]], axis=1)


@jax.jit
def _transpose(tableT):
    return pl.pallas_call(
        _transpose_block,
        grid=(TGRID,),
        in_specs=[pl.BlockSpec((EMBED_DIM, TNV), lambda i: (0, i))],
        out_specs=pl.BlockSpec((TNV // 2, 2 * EMBED_DIM), lambda i: (i, 0)),
        out_shape=jax.ShapeDtypeStruct((PACKV, 2 * EMBED_DIM), jnp.float32),
        compiler_params=pltpu.CompilerParams(
            dimension_semantics=("parallel",)
        ),
        name="cbow_transpose",
    )(tableT)


@jax.jit
def _cbow(idx2d, table2):
    mesh = plsc.VectorSubcoreMesh(
        core_axis_name="c", subcore_axis_name="s", num_cores=NC, num_subcores=NS
    )
    return pl.kernel(
        _cbow_body,
        out_type=jax.ShapeDtypeStruct((BATCH, EMBED_DIM), jnp.float32),
        mesh=mesh,
        scratch_types=[
            pltpu.VMEM((IROWS, IDX_W), jnp.int32),
            pltpu.VMEM((IROWS * IDX_W + LANES,), jnp.int32),
            pltpu.VMEM((IPC, 2 * EMBED_DIM), jnp.float32),
            pltpu.VMEM((CB, EMBED_DIM), jnp.float32),
            pltpu.SemaphoreType.DMA,
        ],
        compiler_params=pltpu.CompilerParams(use_tc_tiling_on_sc=False),
        name="cbow_sc",
    )(idx2d, table2)


def kernel(context_idxs, embeddings):
    idx2d = context_idxs.astype(jnp.int32).reshape(BATCH * CTX // IDX_W, IDX_W)
    table2 = _transpose(embeddings.T)
    return _cbow(idx2d, table2)


# speedup vs baseline: 2.2769x; 1.1202x over previous
import jax
import jax.numpy as jnp
from jax import lax
from jax.experimental import pallas as pl
from jax.experimental.pallas import tpu as pltpu
from jax.experimental.pallas import tpu_sc as plsc

VOCAB = 1000000
EMBED_DIM = 64
BATCH = 16384
CTX = 20

NC = 2
NS = 16
LANES = 16
NW = NC * NS
BPW = BATCH // NW
CB = 32
NCH = BPW // CB
IPC = CB * CTX
IDX_W = 128
NSTR = IPC // IDX_W
VPD = EMBED_DIM // LANES
IROWS = BPW * CTX // IDX_W

TNV = 32768
HB = 14
TGRID = (VOCAB + TNV - 1) // TNV
PACKV = TGRID * TNV // 2


def _cbow_body(idx_hbm, table_hbm, out_hbm, idx_v, rows_v, out_v, sem):
    wid = lax.axis_index("s") * NC + lax.axis_index("c")
    inv_ctx = jnp.float32(1.0 / CTX)

    pltpu.sync_copy(idx_hbm.at[pl.ds(wid * IROWS, IROWS)], idx_v)

    def prep_body(r, _):
        for g in range(IDX_W // LANES):
            sl = pl.ds(g * LANES, LANES)
            v = idx_v[r, sl]
            packed = lax.bitwise_or(
                lax.shift_left(lax.shift_right_logical(v, jnp.int32(HB + 1)), jnp.int32(HB)),
                lax.bitwise_and(v, jnp.int32(TNV // 2 - 1)),
            )
            idx_v[r, sl] = lax.bitwise_or(
                lax.shift_left(packed, jnp.int32(1)),
                lax.bitwise_and(lax.shift_right_logical(v, jnp.int32(HB)), jnp.int32(1)),
            )
        return 0

    lax.fori_loop(0, IROWS, prep_body, 0)

    def chunk_body(c, _):
        copies = [
            pltpu.async_copy(
                table_hbm.at[idx_v.at[c * NSTR + k]],
                rows_v.at[pl.ds(k * IDX_W, IDX_W)],
                sem,
            )
            for k in range(NSTR)
        ]
        for cp in copies:
            cp.wait()

        def item_body(i, _):
            base = i * CTX
            acc = [None] * VPD
            for j in range(CTX):
                for q in range(VPD):
                    x = rows_v[base + j, pl.ds(q * LANES, LANES)]
                    acc[q] = x if acc[q] is None else acc[q] + x
            for q in range(VPD):
                out_v[i, pl.ds(q * LANES, LANES)] = acc[q] * inv_ctx
            return 0

        lax.fori_loop(0, CB, item_body, 0)

        out_row0 = wid * BPW + c * CB
        pltpu.sync_copy(out_v, out_hbm.at[pl.ds(out_row0, CB)])
        return 0

    lax.fori_loop(0, NCH, chunk_body, 0)


def _transpose_block(xT_ref, out_ref):
    xT = xT_ref[...]
    rows = xT.T
    out_ref[...] = jnp.concatenate([rows[: TNV // 2], rows[TNV // 2:]], axis=1)


@jax.jit
def _transpose(tableT):
    return pl.pallas_call(
        _transpose_block,
        grid=(TGRID,),
        in_specs=[pl.BlockSpec((EMBED_DIM, TNV), lambda i: (0, i))],
        out_specs=pl.BlockSpec((TNV // 2, 2 * EMBED_DIM), lambda i: (i, 0)),
        out_shape=jax.ShapeDtypeStruct((PACKV, 2 * EMBED_DIM), jnp.float32),
        compiler_params=pltpu.CompilerParams(
            dimension_semantics=("parallel",)
        ),
        name="cbow_transpose",
    )(tableT)


@jax.jit
def _cbow(idx2d, table2):
    mesh = plsc.VectorSubcoreMesh(
        core_axis_name="c", subcore_axis_name="s", num_cores=NC, num_subcores=NS
    )
    return pl.kernel(
        _cbow_body,
        out_type=jax.ShapeDtypeStruct((BATCH, EMBED_DIM), jnp.float32),
        mesh=mesh,
        scratch_types=[
            pltpu.VMEM((IROWS, IDX_W), jnp.int32),
            pltpu.VMEM((IPC, EMBED_DIM), jnp.float32),
            pltpu.VMEM((CB, EMBED_DIM), jnp.float32),
            pltpu.SemaphoreType.DMA,
        ],
        compiler_params=pltpu.CompilerParams(use_tc_tiling_on_sc=False),
        name="cbow_sc",
    )(idx2d, table2)


def kernel(context_idxs, embeddings):
    idx2d = context_idxs.astype(jnp.int32).reshape(BATCH * CTX // IDX_W, IDX_W)
    table2 = _transpose(embeddings.T)
    return _cbow(idx2d, table2.reshape(2 * PACKV, EMBED_DIM))


# speedup vs baseline: 2.4906x; 1.0938x over previous
import jax
import jax.numpy as jnp
from jax import lax
from jax.experimental import pallas as pl
from jax.experimental.pallas import tpu as pltpu
from jax.experimental.pallas import tpu_sc as plsc

VOCAB = 1000000
EMBED_DIM = 64
BATCH = 16384
CTX = 20

NC = 2
NS = 16
LANES = 16
NW = NC * NS
BPW = BATCH // NW
CB = 32
NCH = BPW // CB
IPC = CB * CTX
IDX_W = 128
NSTR = IPC // IDX_W
VPD = EMBED_DIM // LANES
IROWS = BPW * CTX // IDX_W

TNV = 32768
HB = 14
TGRID = (VOCAB + TNV - 1) // TNV
PACKV = TGRID * TNV // 2


def _cbow_body(idx_hbm, table_hbm, out_hbm, idx_v, rows_v, out_v, sg0, sg1, so0, so1):
    wid = lax.axis_index("s") * NC + lax.axis_index("c")
    inv_ctx = jnp.float32(1.0 / CTX)

    pltpu.sync_copy(idx_hbm.at[pl.ds(wid * IROWS, IROWS)], idx_v)

    def prep_body(r, _):
        for g in range(IDX_W // LANES):
            sl = pl.ds(g * LANES, LANES)
            v = idx_v[r, sl]
            packed = lax.bitwise_or(
                lax.shift_left(lax.shift_right_logical(v, jnp.int32(HB + 1)), jnp.int32(HB)),
                lax.bitwise_and(v, jnp.int32(TNV // 2 - 1)),
            )
            idx_v[r, sl] = lax.bitwise_or(
                lax.shift_left(packed, jnp.int32(1)),
                lax.bitwise_and(lax.shift_right_logical(v, jnp.int32(HB)), jnp.int32(1)),
            )
        return 0

    lax.fori_loop(0, IROWS, prep_body, 0)

    sg = (sg0, sg1)
    so = (so0, so1)

    def issue(c, s):
        return [
            pltpu.async_copy(
                table_hbm.at[idx_v.at[c * NSTR + k]],
                rows_v.at[pl.ds(s * IPC + k * IDX_W, IDX_W)],
                sg[s],
            )
            for k in range(NSTR)
        ]

    gcp = [issue(0, 0), None]
    ocp = [None, None]
    for c in range(NCH):
        s = c & 1
        if c + 1 < NCH:
            gcp[1 - s] = issue(c + 1, 1 - s)
        for cp in gcp[s]:
            cp.wait()
        if ocp[s] is not None:
            ocp[s].wait()

        def item_body(i, _, s=s):
            base = s * IPC + i * CTX
            acc = [None] * VPD
            for j in range(CTX):
                for q in range(VPD):
                    x = rows_v[base + j, pl.ds(q * LANES, LANES)]
                    acc[q] = x if acc[q] is None else acc[q] + x
            for q in range(VPD):
                out_v[s * CB + i, pl.ds(q * LANES, LANES)] = acc[q] * inv_ctx
            return 0

        lax.fori_loop(0, CB, item_body, 0)

        ocp[s] = pltpu.async_copy(
            out_v.at[pl.ds(s * CB, CB)],
            out_hbm.at[pl.ds(wid * BPW + c * CB, CB)],
            so[s],
        )
    ocp[0].wait()
    ocp[1].wait()


def _transpose_block(xT_ref, out_ref):
    xT = xT_ref[...]
    rows = xT.T
    out_ref[...] = jnp.concatenate([rows[: TNV // 2], rows[TNV // 2:]], axis=1)


@jax.jit
def _transpose(tableT):
    return pl.pallas_call(
        _transpose_block,
        grid=(TGRID,),
        in_specs=[pl.BlockSpec((EMBED_DIM, TNV), lambda i: (0, i))],
        out_specs=pl.BlockSpec((TNV // 2, 2 * EMBED_DIM), lambda i: (i, 0)),
        out_shape=jax.ShapeDtypeStruct((PACKV, 2 * EMBED_DIM), jnp.float32),
        compiler_params=pltpu.CompilerParams(
            dimension_semantics=("parallel",)
        ),
        name="cbow_transpose",
    )(tableT)


@jax.jit
def _cbow(idx2d, table2):
    mesh = plsc.VectorSubcoreMesh(
        core_axis_name="c", subcore_axis_name="s", num_cores=NC, num_subcores=NS
    )
    return pl.kernel(
        _cbow_body,
        out_type=jax.ShapeDtypeStruct((BATCH, EMBED_DIM), jnp.float32),
        mesh=mesh,
        scratch_types=[
            pltpu.VMEM((IROWS, IDX_W), jnp.int32),
            pltpu.VMEM((2 * IPC, EMBED_DIM), jnp.float32),
            pltpu.VMEM((2 * CB, EMBED_DIM), jnp.float32),
            pltpu.SemaphoreType.DMA,
            pltpu.SemaphoreType.DMA,
            pltpu.SemaphoreType.DMA,
            pltpu.SemaphoreType.DMA,
        ],
        compiler_params=pltpu.CompilerParams(use_tc_tiling_on_sc=False),
        name="cbow_sc",
    )(idx2d, table2)


def kernel(context_idxs, embeddings):
    idx2d = context_idxs.astype(jnp.int32).reshape(BATCH * CTX // IDX_W, IDX_W)
    table2 = _transpose(embeddings.T)
    return _cbow(idx2d, table2.reshape(2 * PACKV, EMBED_DIM))
